# Initial kernel scaffold; baseline (speedup 1.0000x reference)
import jax
import jax.numpy as jnp
from jax.experimental import pallas as pl


def kernel(emb_vector):
    raise NotImplementedError("write your pallas kernel here")


# speedup vs baseline: 1.0948x; 1.0948x over previous
import functools

import jax
import jax.numpy as jnp
from jax import lax
from jax.experimental import pallas as pl
from jax.experimental.pallas import tpu as pltpu
from jax.experimental.pallas import tpu_sc as plsc

B = 16384
F = 100
D = 32
G = 20
S = 5
L = 16
ROW = F * D
OROW = G * D

NC = 2
NS = 16
NW = NC * NS
BPW = B // NW

C = 4
NCHUNK = BPW // C


def _compute_chunk(in_v, out_v):
    def sample_body(s, carry):
        for g in range(G):
            for h in range(D // L):
                col0 = g * S * D + h * L
                acc = in_v[s, pl.ds(col0, L)]
                for j in range(1, S):
                    acc = acc + in_v[s, pl.ds(col0 + j * D, L)]
                out_v[s, pl.ds(g * D + h * L, L)] = acc * (1.0 / S)
        return carry

    lax.fori_loop(0, C, sample_body, 0)


def _body(in_hbm, out_hbm, in_v0, in_v1, out_v0, out_v1, si0, si1, so0, so1):
    wid = lax.axis_index("s") * NC + lax.axis_index("c")
    base = wid * BPW
    in_bufs = (in_v0, in_v1)
    out_bufs = (out_v0, out_v1)
    in_sems = (si0, si1)
    out_sems = (so0, so1)

    def start_in(k, b):
        pltpu.make_async_copy(
            in_hbm.at[pl.ds(base + k * C, C)], in_bufs[b], in_sems[b]
        ).start()

    def wait_in(b):
        pltpu.make_async_copy(
            in_hbm.at[pl.ds(0, C)], in_bufs[b], in_sems[b]
        ).wait()

    def start_out(k, b):
        pltpu.make_async_copy(
            out_bufs[b], out_hbm.at[pl.ds(base + k * C, C)], out_sems[b]
        ).start()

    def wait_out(b):
        pltpu.make_async_copy(
            out_bufs[b], out_hbm.at[pl.ds(0, C)], out_sems[b]
        ).wait()

    start_in(0, 0)
    start_in(1, 1)

    def loop_body(i, carry):
        for b in range(2):
            k = 2 * i + b
            wait_in(b)

            @pl.when(k >= 2)
            def _():
                wait_out(b)

            _compute_chunk(in_bufs[b], out_bufs[b])
            start_out(k, b)

            @pl.when(k + 2 < NCHUNK)
            def _():
                start_in(k + 2, b)

        return carry

    lax.fori_loop(0, NCHUNK // 2, loop_body, 0)
    wait_out(0)
    wait_out(1)


@jax.jit
def kernel(emb_vector):
    x = emb_vector.reshape(B, ROW)
    mesh = plsc.VectorSubcoreMesh(core_axis_name="c", subcore_axis_name="s")
    out = pl.kernel(
        _body,
        out_type=jax.ShapeDtypeStruct((B, OROW), jnp.float32),
        mesh=mesh,
        scratch_types=[
            pltpu.VMEM((C, ROW), jnp.float32),
            pltpu.VMEM((C, ROW), jnp.float32),
            pltpu.VMEM((C, OROW), jnp.float32),
            pltpu.VMEM((C, OROW), jnp.float32),
            pltpu.SemaphoreType.DMA,
            pltpu.SemaphoreType.DMA,
            pltpu.SemaphoreType.DMA,
            pltpu.SemaphoreType.DMA,
        ],
    )(x)
    return out.reshape(B, G, D)


# speedup vs baseline: 1.0955x; 1.0006x over previous
import functools

import jax
import jax.numpy as jnp
from jax import lax
from jax.experimental import pallas as pl
from jax.experimental.pallas import tpu as pltpu
from jax.experimental.pallas import tpu_sc as plsc

B = 16384
F = 100
D = 32
G = 20
S = 5
L = 16
ROW = F * D
OROW = G * D

NC = 2
NS = 16
NW = NC * NS
BPW = B // NW

C = 4
NCHUNK = BPW // C


def _compute_chunk(in_v, out_v):
    def sample_body(s, carry):
        for g in range(G):
            for h in range(D // L):
                col0 = g * S * D + h * L
                acc = in_v[s, pl.ds(col0, L)]
                for j in range(1, S):
                    acc = acc + in_v[s, pl.ds(col0 + j * D, L)]
                out_v[s, pl.ds(g * D + h * L, L)] = acc * (1.0 / S)
        return carry

    lax.fori_loop(0, C, sample_body, 0)


def _body(in_hbm, out_hbm, in_v0, in_v1, out_v0, out_v1, si0, si1, so0, so1):
    wid = lax.axis_index("s") * NC + lax.axis_index("c")
    base = wid * BPW
    in_bufs = (in_v0, in_v1)
    out_bufs = (out_v0, out_v1)
    in_sems = (si0, si1)
    out_sems = (so0, so1)

    def start_in(k, b):
        pltpu.make_async_copy(
            in_hbm.at[pl.ds(base + k * C, C)], in_bufs[b], in_sems[b]
        ).start()

    def wait_in(b):
        pltpu.make_async_copy(
            in_hbm.at[pl.ds(0, C)], in_bufs[b], in_sems[b]
        ).wait()

    def start_out(k, b):
        pltpu.make_async_copy(
            out_bufs[b], out_hbm.at[pl.ds(base + k * C, C)], out_sems[b]
        ).start()

    def wait_out(b):
        pltpu.make_async_copy(
            out_bufs[b], out_hbm.at[pl.ds(0, C)], out_sems[b]
        ).wait()

    start_in(0, 0)
    start_in(1, 1)

    def loop_body(i, carry):
        for b in range(2):
            k = 2 * i + b
            wait_in(b)

            @pl.when(k >= 2)
            def _():
                wait_out(b)

            _compute_chunk(in_bufs[b], out_bufs[b])
            start_out(k, b)

            @pl.when(k + 2 < NCHUNK)
            def _():
                start_in(k + 2, b)

        return carry

    lax.fori_loop(0, NCHUNK // 2, loop_body, 0)
    wait_out(0)
    wait_out(1)


@jax.jit
def kernel(emb_vector):
    x = emb_vector.reshape(B, ROW)
    mesh = plsc.VectorSubcoreMesh(core_axis_name="c", subcore_axis_name="s")
    out = pl.kernel(
        _body,
        out_type=jax.ShapeDtypeStruct((B, OROW), jnp.float32),
        mesh=mesh,
        compiler_params=pltpu.CompilerParams(use_tc_tiling_on_sc=True),
        scratch_types=[
            pltpu.VMEM((C, ROW), jnp.float32),
            pltpu.VMEM((C, ROW), jnp.float32),
            pltpu.VMEM((C, OROW), jnp.float32),
            pltpu.VMEM((C, OROW), jnp.float32),
            pltpu.SemaphoreType.DMA,
            pltpu.SemaphoreType.DMA,
            pltpu.SemaphoreType.DMA,
            pltpu.SemaphoreType.DMA,
        ],
    )(x)
    return out.reshape(B, G, D)
